# baseline (speedup 1.0000x reference)
import jax
import jax.numpy as jnp
from jax import lax
from jax.experimental import pallas as pl
from jax.experimental.pallas import tpu as pltpu, tpu_sc as plsc
import functools

VOCAB = 13
D = 128
L = 200
B = 4096
NC = 2
NS = 16
LANES = 16

ROWS = B * L
ROWS_PER_W = ROWS // (NC * NS)
CHUNK = 256
N_CHUNKS = ROWS_PER_W // CHUNK
NBUF = 2


def _body(x_hbm, tok_hbm, pos_hbm, out_hbm, *refs):
    f_sh, tok_v, prow_v, blk_v, xall = refs[:5]
    idxs = refs[5:5 + NBUF]
    rows = refs[5 + NBUF:5 + 2 * NBUF]
    xsem = refs[5 + 2 * NBUF]
    gsems = refs[6 + 2 * NBUF:6 + 3 * NBUF]
    ssems = refs[6 + 3 * NBUF:6 + 4 * NBUF]

    s = lax.axis_index("s")
    c = lax.axis_index("c")
    wid = c * NS + s
    wbase = wid * ROWS_PER_W
    iota = lax.iota(jnp.int32, LANES)

    xcopy = pltpu.async_copy(x_hbm.at[pl.ds(wbase, ROWS_PER_W)], xall, xsem)

    pltpu.sync_copy(tok_hbm, tok_v)

    def fill(k, _):
        p = s + NS * k

        @pl.when(p < L)
        def _go():
            pltpu.sync_copy(pos_hbm.at[p], prow_v)
            for v in range(VOCAB):
                for cc in range(D // LANES):
                    sl = pl.ds(cc * LANES, LANES)
                    blk_v[v, sl] = tok_v[v, sl] + prow_v[sl]
            pltpu.sync_copy(blk_v, f_sh.at[pl.ds(p * VOCAB, VOCAB)])
        return 0

    lax.fori_loop(0, (L + NS - 1) // NS, fill, 0)
    xcopy.wait()
    plsc.subcore_barrier()

    def start_g(b, t):
        for cc in range(CHUNK // LANES):
            off = t * CHUNK + cc * LANES
            p16 = lax.rem(iota + (wbase + off), L) * VOCAB
            idxs[b][pl.ds(cc * LANES, LANES)] = xall[pl.ds(off, LANES)] + p16
        pltpu.async_copy(f_sh.at[idxs[b]], rows[b], gsems[b])

    def wait_g(b):
        pltpu.make_async_copy(f_sh.at[idxs[b]], rows[b], gsems[b]).wait()

    def start_s(b, t):
        base = wbase + t * CHUNK
        pltpu.async_copy(rows[b], out_hbm.at[pl.ds(base, CHUNK)], ssems[b])

    def wait_s(b):
        pltpu.make_async_copy(
            rows[b], out_hbm.at[pl.ds(wbase, CHUNK)], ssems[b]).wait()

    for b in range(NBUF):
        start_g(b, b)
    for b in range(NBUF - 1):
        wait_g(b)
        start_s(b, b)

    def outer(t0, _):
        for b in range(NBUF):
            t = t0 * NBUF + b
            wait_s(b)
            start_g(b, t)
            bp = (b - 1) % NBUF
            wait_g(bp)
            start_s(bp, t - 1)
        return 0

    lax.fori_loop(1, N_CHUNKS // NBUF, outer, 0)

    wait_g(NBUF - 1)
    start_s(NBUF - 1, N_CHUNKS - 1)
    for b in range(NBUF):
        wait_s(b)


@jax.jit
def _run(x_flat, token_table, position_embedding):
    mesh = plsc.VectorSubcoreMesh(
        core_axis_name="c", subcore_axis_name="s",
        num_cores=NC, num_subcores=NS)
    return pl.kernel(
        _body,
        out_type=jax.ShapeDtypeStruct((ROWS, D), jnp.float32),
        mesh=mesh,
        scratch_types=[
            pltpu.VMEM_SHARED((L * VOCAB, D), jnp.float32),
            pltpu.VMEM((VOCAB, D), jnp.float32),
            pltpu.VMEM((D,), jnp.float32),
            pltpu.VMEM((VOCAB, D), jnp.float32),
            pltpu.VMEM((ROWS_PER_W,), jnp.int32),
            *[pltpu.VMEM((CHUNK,), jnp.int32) for _ in range(NBUF)],
            *[pltpu.VMEM((CHUNK, D), jnp.float32) for _ in range(NBUF)],
            pltpu.SemaphoreType.DMA,
            *[pltpu.SemaphoreType.DMA for _ in range(2 * NBUF)],
        ],
    )(x_flat, token_table, position_embedding)


def kernel(x, token_table, position_embedding):
    x_flat = x.reshape(-1).astype(jnp.int32)
    out = _run(x_flat, token_table, position_embedding)
    return out.reshape(B, L, D)


# speedup vs baseline: 1.0141x; 1.0141x over previous
import jax
import jax.numpy as jnp
from jax import lax
from jax.experimental import pallas as pl
from jax.experimental.pallas import tpu as pltpu, tpu_sc as plsc
import functools

VOCAB = 13
D = 128
L = 200
B = 4096
NC = 2
NS = 16
LANES = 16

ROWS = B * L
ROWS_PER_W = ROWS // (NC * NS)
CHUNK = 64
N_CHUNKS = ROWS_PER_W // CHUNK
NBUF = 8


def _body(x_hbm, tok_hbm, pos_hbm, out_hbm, *refs):
    f_sh, tok_v, prow_v, blk_v, xall = refs[:5]
    idxs = refs[5:5 + NBUF]
    rows = refs[5 + NBUF:5 + 2 * NBUF]
    xsem = refs[5 + 2 * NBUF]
    gsems = refs[6 + 2 * NBUF:6 + 3 * NBUF]
    ssems = refs[6 + 3 * NBUF:6 + 4 * NBUF]

    s = lax.axis_index("s")
    c = lax.axis_index("c")
    wid = c * NS + s
    wbase = wid * ROWS_PER_W
    iota = lax.iota(jnp.int32, LANES)

    xcopy = pltpu.async_copy(x_hbm.at[pl.ds(wbase, ROWS_PER_W)], xall, xsem)

    pltpu.sync_copy(tok_hbm, tok_v)

    def fill(k, _):
        p = s + NS * k

        @pl.when(p < L)
        def _go():
            pltpu.sync_copy(pos_hbm.at[p], prow_v)
            for v in range(VOCAB):
                for cc in range(D // LANES):
                    sl = pl.ds(cc * LANES, LANES)
                    blk_v[v, sl] = tok_v[v, sl] + prow_v[sl]
            pltpu.sync_copy(blk_v, f_sh.at[pl.ds(p * VOCAB, VOCAB)])
        return 0

    lax.fori_loop(0, (L + NS - 1) // NS, fill, 0)
    xcopy.wait()
    plsc.subcore_barrier()

    def start_g(b, t):
        for cc in range(CHUNK // LANES):
            off = t * CHUNK + cc * LANES
            p16 = lax.rem(iota + (wbase + off), L) * VOCAB
            idxs[b][pl.ds(cc * LANES, LANES)] = xall[pl.ds(off, LANES)] + p16
        pltpu.async_copy(f_sh.at[idxs[b]], rows[b], gsems[b])

    def wait_g(b):
        pltpu.make_async_copy(f_sh.at[idxs[b]], rows[b], gsems[b]).wait()

    def start_s(b, t):
        base = wbase + t * CHUNK
        pltpu.async_copy(rows[b], out_hbm.at[pl.ds(base, CHUNK)], ssems[b])

    def wait_s(b):
        pltpu.make_async_copy(
            rows[b], out_hbm.at[pl.ds(wbase, CHUNK)], ssems[b]).wait()

    for b in range(NBUF):
        start_g(b, b)
    for b in range(NBUF - 1):
        wait_g(b)
        start_s(b, b)

    def outer(t0, _):
        for b in range(NBUF):
            t = t0 * NBUF + b
            wait_s(b)
            start_g(b, t)
            bp = (b - 1) % NBUF
            wait_g(bp)
            start_s(bp, t - 1)
        return 0

    lax.fori_loop(1, N_CHUNKS // NBUF, outer, 0)

    wait_g(NBUF - 1)
    start_s(NBUF - 1, N_CHUNKS - 1)
    for b in range(NBUF):
        wait_s(b)


@jax.jit
def _run(x_flat, token_table, position_embedding):
    mesh = plsc.VectorSubcoreMesh(
        core_axis_name="c", subcore_axis_name="s",
        num_cores=NC, num_subcores=NS)
    return pl.kernel(
        _body,
        out_type=jax.ShapeDtypeStruct((ROWS, D), jnp.float32),
        mesh=mesh,
        scratch_types=[
            pltpu.VMEM_SHARED((L * VOCAB, D), jnp.float32),
            pltpu.VMEM((VOCAB, D), jnp.float32),
            pltpu.VMEM((D,), jnp.float32),
            pltpu.VMEM((VOCAB, D), jnp.float32),
            pltpu.VMEM((ROWS_PER_W,), jnp.int32),
            *[pltpu.VMEM((CHUNK,), jnp.int32) for _ in range(NBUF)],
            *[pltpu.VMEM((CHUNK, D), jnp.float32) for _ in range(NBUF)],
            pltpu.SemaphoreType.DMA,
            *[pltpu.SemaphoreType.DMA for _ in range(2 * NBUF)],
        ],
    )(x_flat, token_table, position_embedding)


def kernel(x, token_table, position_embedding):
    x_flat = x.reshape(-1).astype(jnp.int32)
    out = _run(x_flat, token_table, position_embedding)
    return out.reshape(B, L, D)


# speedup vs baseline: 1.0191x; 1.0049x over previous
import jax
import jax.numpy as jnp
from jax import lax
from jax.experimental import pallas as pl
from jax.experimental.pallas import tpu as pltpu, tpu_sc as plsc
import functools

VOCAB = 13
D = 128
L = 200
B = 4096
NC = 2
NS = 16
LANES = 16

ROWS = B * L
ROWS_PER_W = ROWS // (NC * NS)
CHUNK = 64
N_CHUNKS = ROWS_PER_W // CHUNK
NBUF = 8


def _body(x_hbm, tok_hbm, pos_hbm, out_hbm, *refs):
    f_sh, prow_v, xall = refs[:3]
    idxs = refs[3:3 + NBUF]
    rows = refs[3 + NBUF:3 + 2 * NBUF]
    xsem = refs[3 + 2 * NBUF]
    gsems = refs[4 + 2 * NBUF:4 + 3 * NBUF]
    ssems = refs[4 + 3 * NBUF:4 + 4 * NBUF]

    s = lax.axis_index("s")
    c = lax.axis_index("c")
    wid = c * NS + s
    wbase = wid * ROWS_PER_W
    iota = lax.iota(jnp.int32, LANES)

    xcopy = pltpu.async_copy(x_hbm.at[pl.ds(wbase, ROWS_PER_W)], xall, xsem)

    @pl.when(s < VOCAB)
    def _fill():
        pltpu.sync_copy(tok_hbm.at[s], prow_v)
        for r in range(4):
            pltpu.async_copy(pos_hbm.at[pl.ds(r * 64, 64)], rows[r], gsems[r])
        for r in range(4):
            pltpu.make_async_copy(
                pos_hbm.at[pl.ds(r * 64, 64)], rows[r], gsems[r]).wait()
        vbase = s * L

        for r in range(3):
            def piece(i, _, r=r):
                for cc in range(D // LANES):
                    sl = pl.ds(cc * LANES, LANES)
                    rows[r + 4][i, sl] = rows[r][i, sl] + prow_v[sl]
                return 0

            lax.fori_loop(0, 64, piece, 0)
            pltpu.async_copy(
                rows[r + 4], f_sh.at[pl.ds(vbase + r * 64, 64)], ssems[r])

        def piece8(i, _):
            for cc in range(D // LANES):
                sl = pl.ds(cc * LANES, LANES)
                rows[7][i, sl] = rows[3][i, sl] + prow_v[sl]
            return 0

        lax.fori_loop(0, 8, piece8, 0)
        pltpu.async_copy(rows[7].at[pl.ds(0, 8)],
                         f_sh.at[pl.ds(vbase + 192, 8)], ssems[3])
        for r in range(3):
            pltpu.make_async_copy(
                rows[r + 4], f_sh.at[pl.ds(vbase + r * 64, 64)],
                ssems[r]).wait()
        pltpu.make_async_copy(rows[7].at[pl.ds(0, 8)],
                              f_sh.at[pl.ds(vbase + 192, 8)], ssems[3]).wait()

    xcopy.wait()
    plsc.subcore_barrier()

    def start_g(b, t):
        for cc in range(CHUNK // LANES):
            off = t * CHUNK + cc * LANES
            p16 = lax.rem(iota + (wbase + off), L)
            idxs[b][pl.ds(cc * LANES, LANES)] = (
                xall[pl.ds(off, LANES)] * L + p16)
        pltpu.async_copy(f_sh.at[idxs[b]], rows[b], gsems[b])

    def wait_g(b):
        pltpu.make_async_copy(f_sh.at[idxs[b]], rows[b], gsems[b]).wait()

    def start_s(b, t):
        base = wbase + t * CHUNK
        pltpu.async_copy(rows[b], out_hbm.at[pl.ds(base, CHUNK)], ssems[b])

    def wait_s(b):
        pltpu.make_async_copy(
            rows[b], out_hbm.at[pl.ds(wbase, CHUNK)], ssems[b]).wait()

    for b in range(NBUF):
        start_g(b, b)
    for b in range(NBUF - 1):
        wait_g(b)
        start_s(b, b)

    def outer(t0, _):
        for b in range(NBUF):
            t = t0 * NBUF + b
            wait_s(b)
            start_g(b, t)
            bp = (b - 1) % NBUF
            wait_g(bp)
            start_s(bp, t - 1)
        return 0

    lax.fori_loop(1, N_CHUNKS // NBUF, outer, 0)

    wait_g(NBUF - 1)
    start_s(NBUF - 1, N_CHUNKS - 1)
    for b in range(NBUF):
        wait_s(b)


@jax.jit
def _run(x_flat, token_table, position_embedding):
    mesh = plsc.VectorSubcoreMesh(
        core_axis_name="c", subcore_axis_name="s",
        num_cores=NC, num_subcores=NS)
    return pl.kernel(
        _body,
        out_type=jax.ShapeDtypeStruct((ROWS, D), jnp.float32),
        mesh=mesh,
        scratch_types=[
            pltpu.VMEM_SHARED((L * VOCAB, D), jnp.float32),
            pltpu.VMEM((D,), jnp.float32),
            pltpu.VMEM((ROWS_PER_W,), jnp.int32),
            *[pltpu.VMEM((CHUNK,), jnp.int32) for _ in range(NBUF)],
            *[pltpu.VMEM((CHUNK, D), jnp.float32) for _ in range(NBUF)],
            pltpu.SemaphoreType.DMA,
            *[pltpu.SemaphoreType.DMA for _ in range(2 * NBUF)],
        ],
    )(x_flat, token_table, position_embedding)


def kernel(x, token_table, position_embedding):
    x_flat = x.reshape(-1).astype(jnp.int32)
    out = _run(x_flat, token_table, position_embedding)
    return out.reshape(B, L, D)


# speedup vs baseline: 1.0605x; 1.0406x over previous
import jax
import jax.numpy as jnp
from jax import lax
from jax.experimental import pallas as pl
from jax.experimental.pallas import tpu as pltpu, tpu_sc as plsc
import functools

VOCAB = 13
D = 128
L = 200
B = 4096
NC = 2
NS = 16
LANES = 16

ROWS = B * L
ROWS_PER_W = ROWS // (NC * NS)
CHUNK = 64
N_CHUNKS = ROWS_PER_W // CHUNK
NBUF = 8
LP = 208


def _body(x_hbm, tok_hbm, pos_hbm, out_hbm, *refs):
    f_sh, prow_v, xall, tokrep, idx16 = refs[:5]
    idxs = refs[5:5 + NBUF]
    rows = refs[5 + NBUF:5 + 2 * NBUF]
    xsem = refs[5 + 2 * NBUF]
    gsems = refs[6 + 2 * NBUF:6 + 3 * NBUF]
    ssems = refs[6 + 3 * NBUF:6 + 4 * NBUF]

    s = lax.axis_index("s")
    c = lax.axis_index("c")
    wid = c * NS + s
    wbase = wid * ROWS_PER_W
    iota = lax.iota(jnp.int32, LANES)

    xcopy = pltpu.async_copy(x_hbm.at[pl.ds(wbase, ROWS_PER_W)], xall, xsem)

    @pl.when(s < VOCAB)
    def _fill():
        vbase = s * LP
        pcopy = pltpu.async_copy(
            pos_hbm.at[pl.ds(0, LP)], f_sh.at[pl.ds(vbase, LP)], gsems[0])
        pltpu.sync_copy(tok_hbm.at[s], prow_v)

        def rep(i, _):
            for cc in range(D // LANES):
                sl = pl.ds(cc * LANES, LANES)
                tokrep[i, sl] = prow_v[sl]
            return 0

        lax.fori_loop(0, 64, rep, 0)
        for r in range(3):
            for k in range(4):
                idxs[r][pl.ds(k * LANES, LANES)] = (
                    vbase + r * 64 + k * LANES + iota)
        idx16[pl.ds(0, LANES)] = vbase + 192 + iota
        pcopy.wait()
        for r in range(3):
            pltpu.sync_copy(tokrep, f_sh.at[idxs[r]], add=True)
        pltpu.sync_copy(tokrep.at[pl.ds(0, LANES)], f_sh.at[idx16], add=True)

    xcopy.wait()
    plsc.subcore_barrier()

    def start_g(b, t):
        for cc in range(CHUNK // LANES):
            off = t * CHUNK + cc * LANES
            p16 = lax.rem(iota + (wbase + off), L)
            idxs[b][pl.ds(cc * LANES, LANES)] = (
                xall[pl.ds(off, LANES)] * LP + p16)
        pltpu.async_copy(f_sh.at[idxs[b]], rows[b], gsems[b])

    def wait_g(b):
        pltpu.make_async_copy(f_sh.at[idxs[b]], rows[b], gsems[b]).wait()

    def start_s(b, t):
        base = wbase + t * CHUNK
        pltpu.async_copy(rows[b], out_hbm.at[pl.ds(base, CHUNK)], ssems[b])

    def wait_s(b):
        pltpu.make_async_copy(
            rows[b], out_hbm.at[pl.ds(wbase, CHUNK)], ssems[b]).wait()

    for b in range(NBUF):
        start_g(b, b)
    for b in range(NBUF - 1):
        wait_g(b)
        start_s(b, b)

    def outer(t0, _):
        for b in range(NBUF):
            t = t0 * NBUF + b
            wait_s(b)
            start_g(b, t)
            bp = (b - 1) % NBUF
            wait_g(bp)
            start_s(bp, t - 1)
        return 0

    lax.fori_loop(1, N_CHUNKS // NBUF, outer, 0)

    wait_g(NBUF - 1)
    start_s(NBUF - 1, N_CHUNKS - 1)
    for b in range(NBUF):
        wait_s(b)


@jax.jit
def _run(x_flat, token_table, position_embedding):
    mesh = plsc.VectorSubcoreMesh(
        core_axis_name="c", subcore_axis_name="s",
        num_cores=NC, num_subcores=NS)
    return pl.kernel(
        _body,
        out_type=jax.ShapeDtypeStruct((ROWS, D), jnp.float32),
        mesh=mesh,
        scratch_types=[
            pltpu.VMEM_SHARED((LP * VOCAB, D), jnp.float32),
            pltpu.VMEM((D,), jnp.float32),
            pltpu.VMEM((ROWS_PER_W,), jnp.int32),
            pltpu.VMEM((64, D), jnp.float32),
            pltpu.VMEM((LANES,), jnp.int32),
            *[pltpu.VMEM((CHUNK,), jnp.int32) for _ in range(NBUF)],
            *[pltpu.VMEM((CHUNK, D), jnp.float32) for _ in range(NBUF)],
            pltpu.SemaphoreType.DMA,
            *[pltpu.SemaphoreType.DMA for _ in range(2 * NBUF)],
        ],
    )(x_flat, token_table, position_embedding)


def kernel(x, token_table, position_embedding):
    x_flat = x.reshape(-1).astype(jnp.int32)
    out = _run(x_flat, token_table, position_embedding)
    return out.reshape(B, L, D)
